# baseline (speedup 1.0000x reference)
import functools
import math

import jax
import jax.numpy as jnp
from jax import lax
from jax.experimental import pallas as pl
from jax.experimental.pallas import tpu as pltpu
from jax.experimental.pallas import tpu_sc as plsc

EMBED = 64
SCALE = math.sqrt(EMBED)

NC = 2
NS = 16
NW = NC * NS
LANES = 16

CHUNK = 128


def _build_lookup(batch):
    per_w = batch // NW
    n_chunk = per_w // CHUNK
    mesh = plsc.VectorSubcoreMesh(core_axis_name="c", subcore_axis_name="s")

    @functools.partial(
        pl.kernel,
        mesh=mesh,
        out_type=jax.ShapeDtypeStruct((batch, EMBED), jnp.float32),
        compiler_params=pltpu.CompilerParams(use_tc_tiling_on_sc=False),
        scratch_types=[
            pltpu.VMEM((n_chunk, CHUNK), jnp.int32),
            pltpu.VMEM((CHUNK, EMBED), jnp.float32),
            pltpu.SemaphoreType.DMA,
        ],
    )
    def lookup(idx_hbm, table_hbm, out_hbm, idx_v, rows_v, gsem):
        wid = lax.axis_index("s") * NC + lax.axis_index("c")
        pltpu.sync_copy(idx_hbm.at[wid], idx_v)
        base = wid * per_w

        def chunk_body(c, carry):
            pltpu.async_copy(table_hbm.at[idx_v.at[c]], rows_v, gsem).wait()

            def scale_row(i, carry2):
                for j in range(EMBED // LANES):
                    sl = pl.ds(j * LANES, LANES)
                    rows_v[i, sl] = rows_v[i, sl] * SCALE
                return carry2

            lax.fori_loop(0, CHUNK, scale_row, 0, unroll=2)
            start = pl.multiple_of(base + c * CHUNK, CHUNK)
            pltpu.sync_copy(rows_v, out_hbm.at[pl.ds(start, CHUNK)])
            return carry

        lax.fori_loop(0, n_chunk, chunk_body, 0)

    return lookup


def kernel(inputs, table):
    b0, b1 = inputs.shape
    batch = b0 * b1
    idx = inputs.reshape(NW, (batch // NW) // CHUNK, CHUNK).astype(jnp.int32)
    out = _build_lookup(batch)(idx, table)
    return out.reshape(b0, b1, EMBED)


# speedup vs baseline: 1.1628x; 1.1628x over previous
import functools
import math

import jax
import jax.numpy as jnp
from jax import lax
from jax.experimental import pallas as pl
from jax.experimental.pallas import tpu as pltpu
from jax.experimental.pallas import tpu_sc as plsc

EMBED = 64
SCALE = math.sqrt(EMBED)

NC = 2
NS = 16
NW = NC * NS
LANES = 16

CHUNK = 128
NBUF = 5
PREF = 3


def _build_lookup(batch):
    per_w = batch // NW
    n_chunk = per_w // CHUNK
    assert n_chunk % NBUF == 0
    mesh = plsc.VectorSubcoreMesh(core_axis_name="c", subcore_axis_name="s")

    @functools.partial(
        pl.kernel,
        mesh=mesh,
        out_type=jax.ShapeDtypeStruct((batch, EMBED), jnp.float32),
        compiler_params=pltpu.CompilerParams(use_tc_tiling_on_sc=False),
        scratch_types=(
            [pltpu.VMEM((n_chunk, CHUNK), jnp.int32)]
            + [pltpu.VMEM((CHUNK, EMBED), jnp.float32) for _ in range(NBUF)]
            + [pltpu.SemaphoreType.DMA for _ in range(2 * NBUF)]
        ),
    )
    def lookup(idx_hbm, table_hbm, out_hbm, idx_v, *bufs_and_sems):
        bufs = bufs_and_sems[:NBUF]
        gsem = bufs_and_sems[NBUF : 2 * NBUF]
        ssem = bufs_and_sems[2 * NBUF :]

        wid = lax.axis_index("s") * NC + lax.axis_index("c")
        pltpu.sync_copy(idx_hbm.at[wid], idx_v)
        base = wid * per_w

        def gather_start(c, b):
            pltpu.async_copy(table_hbm.at[idx_v.at[c]], bufs[b], gsem[b])

        def gather_wait(c, b):
            pltpu.make_async_copy(
                table_hbm.at[idx_v.at[c]], bufs[b], gsem[b]
            ).wait()

        def out_slice(c):
            start = pl.multiple_of(base + c * CHUNK, CHUNK)
            return out_hbm.at[pl.ds(start, CHUNK)]

        def store_start(c, b):
            pltpu.async_copy(bufs[b], out_slice(c), ssem[b])

        def store_wait(c, b):
            pltpu.make_async_copy(bufs[b], out_slice(c), ssem[b]).wait()

        for c in range(PREF):
            gather_start(c, c)

        def outer(i, carry):
            for b in range(NBUF):
                c = i * NBUF + b
                gather_wait(c, b)

                buf = bufs[b]

                @plsc.parallel_loop(0, CHUNK, unroll=8)
                def _scale(r):
                    for j in range(EMBED // LANES):
                        sl = pl.ds(j * LANES, LANES)
                        buf[r, sl] = buf[r, sl] * SCALE

                store_start(c, b)

                bt = (b + PREF) % NBUF
                ct = c + PREF

                @pl.when(ct < n_chunk)
                def _prefetch():
                    @pl.when(c >= NBUF - PREF)
                    def _drain():
                        store_wait(ct - NBUF, bt)

                    gather_start(ct, bt)

            return carry

        lax.fori_loop(0, n_chunk // NBUF, outer, 0)

        for b in range(NBUF):
            store_wait(n_chunk - NBUF + b, b)

    return lookup


def kernel(inputs, table):
    b0, b1 = inputs.shape
    batch = b0 * b1
    idx = inputs.reshape(NW, (batch // NW) // CHUNK, CHUNK).astype(jnp.int32)
    out = _build_lookup(batch)(idx, table)
    return out.reshape(b0, b1, EMBED)
